# baseline (speedup 1.0000x reference)
import numpy as np
import jax
import jax.numpy as jnp
from jax.experimental import pallas as pl

_K = 128
_N = 256
_HIDDEN = 512
_B = 1024
_THRESH = 0.5
_RATE = _K / _N


def _build_info_mask():
    m = int(np.log2(_N))
    z = np.array([0.5], dtype=np.float64)
    for _ in range(m):
        z = np.concatenate([2.0 * z - z * z, z * z])
    order = np.argsort(z, kind="stable")
    mask = np.zeros(_N, dtype=bool)
    mask[order[:_K]] = True
    return mask


_INFO_MASK = _build_info_mask()
_FROZEN = ~_INFO_MASK
_INFO_IDX = np.where(_INFO_MASK)[0]


def _encode_rows(mat):
    n = mat.shape[1]
    if n == 1:
        return mat
    h = n // 2
    a = _encode_rows(mat[:, :h])
    b = _encode_rows(mat[:, h:])
    return np.concatenate([a ^ b, b], axis=1)


_G = _encode_rows(np.eye(_N, dtype=np.int64))
_G_INFO = _G[_INFO_IDX, :].astype(np.float32)
_G_UHAT = _G[:, _INFO_IDX].astype(np.float32)


def _mod2(v):
    return v - 2.0 * jnp.floor(v * 0.5)


def _dot_t(a, b):
    return jax.lax.dot_general(a, b, (((0,), (0,)), ((), ())),
                               preferred_element_type=jnp.float32)


_SIGN = np.uint32(0x80000000)


def _u32(v):
    return jax.lax.bitcast_convert_type(v, jnp.uint32)


def _f32(v):
    return jax.lax.bitcast_convert_type(v, jnp.float32)


def _decode_x(llr, fr):
    if fr.all():
        return jnp.zeros(llr.shape, jnp.uint32)
    if not fr.any():
        return _u32(llr) & _SIGN
    h = fr.shape[0] // 2
    la, lb = llr[:h], llr[h:]
    lf, rf = fr[:h], fr[h:]
    if lf.all():
        x2 = _decode_x(la + lb, rf)
        return jnp.concatenate([x2, x2], axis=0)
    l1 = _f32((_u32(la) ^ _u32(lb)) & _SIGN
              | _u32(jnp.minimum(jnp.abs(la), jnp.abs(lb))))
    x1 = _decode_x(l1, lf)
    l2 = lb + _f32(_u32(la) ^ x1)
    x2 = _decode_x(l2, rf)
    return jnp.concatenate([x1 ^ x2, x2], axis=0)


def _fused_body(u_ref, a_ref, nr_ref, ni_ref, ebno_ref,
                w1_ref, b1_ref, w2x_ref, b2x_ref, w2p_ref, b2p_ref,
                gi_ref, gu_ref, id_ref,
                ct_ref, uh_ref, ch_ref, pa_ref, ah_ref, uo_ref, ao_ref):
    uo_ref[...] = u_ref[...]
    ao_ref[...] = a_ref[...]
    a_col = jnp.transpose(a_ref[...])

    ct = jax.lax.dot_general(u_ref[...], gi_ref[...], (((1,), (0,)), ((), ())),
                             preferred_element_type=jnp.float32)
    ct = _mod2(ct)
    ct_ref[...] = ct

    no = 1.0 / (jnp.exp((ebno_ref[0] / 10.0)
                        * np.float32(np.log(10.0))) * _RATE)
    s = jnp.sqrt(no / 2.0)
    y_r = (1.0 - 2.0 * ct) * a_col + nr_ref[...] * s
    y_i = ni_ref[...] * s
    y_ri = jnp.concatenate([y_r, y_i], axis=1)

    h = jnp.maximum(
        jax.lax.dot_general(y_ri, w1_ref[...], (((1,), (0,)), ((), ())),
                            preferred_element_type=jnp.float32)
        + b1_ref[...][None, :],
        0.0)
    logit_t = jax.lax.dot_general(jnp.transpose(w2p_ref[...]), h,
                                  (((0,), (1,)), ((), ())),
                                  preferred_element_type=jnp.float32)
    logit_t = logit_t + b2p_ref[0]
    p = jax.nn.sigmoid(logit_t)
    pa_ref[...] = p
    ah = (p > _THRESH).astype(jnp.float32)
    ah_ref[...] = ah

    yhr_t = jax.lax.dot_general(w2x_ref[...], h, (((0,), (1,)), ((), ())),
                                preferred_element_type=jnp.float32)
    yhr_t = yhr_t + jnp.transpose(b2x_ref[...][None, 0:_N])
    llr_t = 4.0 * yhr_t / no

    xs_root = _decode_x(llr_t, _FROZEN)
    x_root = (xs_root >> 31).astype(jnp.float32)

    xm = x_root * ah
    uh_ref[...] = _mod2(
        jax.lax.dot_general(xm, gu_ref[...], (((0,), (0,)), ((), ())),
                            preferred_element_type=jnp.float32))
    ch_ref[...] = jax.lax.dot_general(xm, id_ref[...], (((0,), (0,)), ((), ())),
                                      preferred_element_type=jnp.float32)


def kernel(u, a_true, noise_r, noise_i, ebno_db, W1, b1, W2x, b2x, W2p, b2p):
    out_shapes = [
        jax.ShapeDtypeStruct((_B, _N), jnp.float32),
        jax.ShapeDtypeStruct((_B, _K), jnp.float32),
        jax.ShapeDtypeStruct((_B, _N), jnp.float32),
        jax.ShapeDtypeStruct((1, _B), jnp.float32),
        jax.ShapeDtypeStruct((1, _B), jnp.float32),
        jax.ShapeDtypeStruct((_B, _K), jnp.float32),
        jax.ShapeDtypeStruct((1, _B), jnp.float32),
    ]
    ct, uh, ch, pa, ah, uo, ao = pl.pallas_call(
        _fused_body,
        grid=(2,),
        in_specs=[
            pl.BlockSpec((_B // 2, _K), lambda i: (i, 0)),
            pl.BlockSpec((1, _B // 2), lambda i: (0, i)),
            pl.BlockSpec((_B // 2, _N), lambda i: (i, 0)),
            pl.BlockSpec((_B // 2, _N), lambda i: (i, 0)),
            pl.BlockSpec((1,), lambda i: (0,)),
            pl.BlockSpec((2 * _N, _HIDDEN), lambda i: (0, 0)),
            pl.BlockSpec((_HIDDEN,), lambda i: (0,)),
            pl.BlockSpec((_HIDDEN, _N), lambda i: (0, 0)),
            pl.BlockSpec((2 * _N,), lambda i: (0,)),
            pl.BlockSpec((1, _HIDDEN), lambda i: (0, 0)),
            pl.BlockSpec((1,), lambda i: (0,)),
            pl.BlockSpec((_K, _N), lambda i: (0, 0)),
            pl.BlockSpec((_N, _K), lambda i: (0, 0)),
            pl.BlockSpec((_N, _N), lambda i: (0, 0)),
        ],
        out_specs=[
            pl.BlockSpec((_B // 2, _N), lambda i: (i, 0)),
            pl.BlockSpec((_B // 2, _K), lambda i: (i, 0)),
            pl.BlockSpec((_B // 2, _N), lambda i: (i, 0)),
            pl.BlockSpec((1, _B // 2), lambda i: (0, i)),
            pl.BlockSpec((1, _B // 2), lambda i: (0, i)),
            pl.BlockSpec((_B // 2, _K), lambda i: (i, 0)),
            pl.BlockSpec((1, _B // 2), lambda i: (0, i)),
        ],
        out_shape=out_shapes,
    )(
        u,
        a_true.reshape(1, _B),
        noise_r,
        noise_i,
        ebno_db.reshape(1),
        W1,
        b1,
        W2x,
        b2x,
        W2p.reshape(1, _HIDDEN),
        b2p,
        jnp.asarray(_G_INFO),
        jnp.asarray(_G_UHAT),
        jnp.asarray(np.eye(_N, dtype=np.float32)),
    )
    return (uo, uh, ct, ch, ao.reshape(_B, 1), pa.reshape(_B, 1),
            ah.reshape(_B, 1))


# speedup vs baseline: 1.0876x; 1.0876x over previous
import numpy as np
import jax
import jax.numpy as jnp
from jax.experimental import pallas as pl

_K = 128
_N = 256
_HIDDEN = 512
_B = 1024
_THRESH = 0.5
_RATE = _K / _N


def _build_info_mask():
    m = int(np.log2(_N))
    z = np.array([0.5], dtype=np.float64)
    for _ in range(m):
        z = np.concatenate([2.0 * z - z * z, z * z])
    order = np.argsort(z, kind="stable")
    mask = np.zeros(_N, dtype=bool)
    mask[order[:_K]] = True
    return mask


_INFO_MASK = _build_info_mask()
_FROZEN = ~_INFO_MASK
_INFO_IDX = np.where(_INFO_MASK)[0]


def _encode_rows(mat):
    n = mat.shape[1]
    if n == 1:
        return mat
    h = n // 2
    a = _encode_rows(mat[:, :h])
    b = _encode_rows(mat[:, h:])
    return np.concatenate([a ^ b, b], axis=1)


_G = _encode_rows(np.eye(_N, dtype=np.int64))
_G_INFO = _G[_INFO_IDX, :].astype(np.float32)
_G_UHAT = _G[:, _INFO_IDX].astype(np.float32)


def _mod2(v):
    return v - 2.0 * jnp.floor(v * 0.5)


def _dot_t(a, b):
    return jax.lax.dot_general(a, b, (((0,), (0,)), ((), ())),
                               preferred_element_type=jnp.float32)


_SIGN = np.uint32(0x80000000)


def _u32(v):
    return jax.lax.bitcast_convert_type(v, jnp.uint32)


def _f32(v):
    return jax.lax.bitcast_convert_type(v, jnp.float32)


def _decode_x(llr, fr):
    if fr.all():
        return jnp.zeros(llr.shape, jnp.uint32)
    if not fr.any():
        return _u32(llr) & _SIGN
    h = fr.shape[0] // 2
    la, lb = llr[:h], llr[h:]
    lf, rf = fr[:h], fr[h:]
    if lf.all():
        x2 = _decode_x(la + lb, rf)
        return jnp.concatenate([x2, x2], axis=0)
    l1 = _f32((_u32(la) ^ _u32(lb)) & _SIGN
              | _u32(jnp.minimum(jnp.abs(la), jnp.abs(lb))))
    x1 = _decode_x(l1, lf)
    l2 = lb + _f32(_u32(la) ^ x1)
    x2 = _decode_x(l2, rf)
    return jnp.concatenate([x1 ^ x2, x2], axis=0)


def _fused_body(u_ref, a_ref, nr_ref, ni_ref, ebno_ref,
                w1_ref, b1_ref, w2x_ref, b2x_ref, w2p_ref, b2p_ref,
                gi_ref, gu_ref, id_ref,
                ct_ref, uh_ref, ch_ref, pa_ref, ah_ref, uo_ref, ao_ref):
    uo_ref[...] = u_ref[...]
    ao_ref[...] = a_ref[...]
    a_col = jnp.transpose(a_ref[...])

    ct = jax.lax.dot_general(u_ref[...], gi_ref[...], (((1,), (0,)), ((), ())),
                             preferred_element_type=jnp.float32)
    ct = _mod2(ct)
    ct_ref[...] = ct

    no = 1.0 / (jnp.exp((ebno_ref[0] / 10.0)
                        * np.float32(np.log(10.0))) * _RATE)
    s = jnp.sqrt(no / 2.0)
    y_r = (1.0 - 2.0 * ct) * a_col + nr_ref[...] * s
    y_i = ni_ref[...] * s
    y_ri = jnp.concatenate([y_r, y_i], axis=1)

    h = jnp.maximum(
        jax.lax.dot_general(y_ri, w1_ref[...], (((1,), (0,)), ((), ())),
                            preferred_element_type=jnp.float32)
        + b1_ref[...][None, :],
        0.0)
    logit_t = jax.lax.dot_general(jnp.transpose(w2p_ref[...]), h,
                                  (((0,), (1,)), ((), ())),
                                  preferred_element_type=jnp.float32)
    logit_t = logit_t + b2p_ref[0]
    p = jax.nn.sigmoid(logit_t)
    pa_ref[...] = p
    ah = (p > _THRESH).astype(jnp.float32)
    ah_ref[...] = ah

    yhr_t = jax.lax.dot_general(w2x_ref[...], h, (((0,), (1,)), ((), ())),
                                preferred_element_type=jnp.float32)
    yhr_t = yhr_t + jnp.transpose(b2x_ref[...][None, 0:_N])
    llr_t = 4.0 * yhr_t / no

    xs_root = _decode_x(llr_t, _FROZEN)
    x_root = (xs_root >> 31).astype(jnp.float32)

    xm = x_root * ah
    uh_ref[...] = _mod2(
        jax.lax.dot_general(xm, gu_ref[...], (((0,), (0,)), ((), ())),
                            preferred_element_type=jnp.float32))
    ch_ref[...] = jax.lax.dot_general(xm, id_ref[...], (((0,), (0,)), ((), ())),
                                      preferred_element_type=jnp.float32)


def kernel(u, a_true, noise_r, noise_i, ebno_db, W1, b1, W2x, b2x, W2p, b2p):
    out_shapes = [
        jax.ShapeDtypeStruct((_B, _N), jnp.float32),
        jax.ShapeDtypeStruct((_B, _K), jnp.float32),
        jax.ShapeDtypeStruct((_B, _N), jnp.float32),
        jax.ShapeDtypeStruct((1, _B), jnp.float32),
        jax.ShapeDtypeStruct((1, _B), jnp.float32),
        jax.ShapeDtypeStruct((_B, _K), jnp.float32),
        jax.ShapeDtypeStruct((1, _B), jnp.float32),
    ]
    ct, uh, ch, pa, ah, uo, ao = pl.pallas_call(
        _fused_body,
        grid=(1,),
        in_specs=[
            pl.BlockSpec((_B, _K), lambda i: (0, 0)),
            pl.BlockSpec((1, _B), lambda i: (0, 0)),
            pl.BlockSpec((_B, _N), lambda i: (0, 0)),
            pl.BlockSpec((_B, _N), lambda i: (0, 0)),
            pl.BlockSpec((1,), lambda i: (0,)),
            pl.BlockSpec((2 * _N, _HIDDEN), lambda i: (0, 0)),
            pl.BlockSpec((_HIDDEN,), lambda i: (0,)),
            pl.BlockSpec((_HIDDEN, _N), lambda i: (0, 0)),
            pl.BlockSpec((2 * _N,), lambda i: (0,)),
            pl.BlockSpec((1, _HIDDEN), lambda i: (0, 0)),
            pl.BlockSpec((1,), lambda i: (0,)),
            pl.BlockSpec((_K, _N), lambda i: (0, 0)),
            pl.BlockSpec((_N, _K), lambda i: (0, 0)),
            pl.BlockSpec((_N, _N), lambda i: (0, 0)),
        ],
        out_specs=[
            pl.BlockSpec((_B, _N), lambda i: (0, 0)),
            pl.BlockSpec((_B, _K), lambda i: (0, 0)),
            pl.BlockSpec((_B, _N), lambda i: (0, 0)),
            pl.BlockSpec((1, _B), lambda i: (0, 0)),
            pl.BlockSpec((1, _B), lambda i: (0, 0)),
            pl.BlockSpec((_B, _K), lambda i: (0, 0)),
            pl.BlockSpec((1, _B), lambda i: (0, 0)),
        ],
        out_shape=out_shapes,
    )(
        u,
        a_true.reshape(1, _B),
        noise_r,
        noise_i,
        ebno_db.reshape(1),
        W1,
        b1,
        W2x,
        b2x,
        W2p.reshape(1, _HIDDEN),
        b2p,
        jnp.asarray(_G_INFO),
        jnp.asarray(_G_UHAT),
        jnp.asarray(np.eye(_N, dtype=np.float32)),
    )
    return (uo, uh, ct, ch, ao.reshape(_B, 1), pa.reshape(_B, 1),
            ah.reshape(_B, 1))


# speedup vs baseline: 1.0894x; 1.0017x over previous
import numpy as np
import jax
import jax.numpy as jnp
from jax.experimental import pallas as pl

_K = 128
_N = 256
_HIDDEN = 512
_B = 1024
_THRESH = 0.5
_RATE = _K / _N


def _build_info_mask():
    m = int(np.log2(_N))
    z = np.array([0.5], dtype=np.float64)
    for _ in range(m):
        z = np.concatenate([2.0 * z - z * z, z * z])
    order = np.argsort(z, kind="stable")
    mask = np.zeros(_N, dtype=bool)
    mask[order[:_K]] = True
    return mask


_INFO_MASK = _build_info_mask()
_FROZEN = ~_INFO_MASK
_INFO_IDX = np.where(_INFO_MASK)[0]


def _encode_rows(mat):
    n = mat.shape[1]
    if n == 1:
        return mat
    h = n // 2
    a = _encode_rows(mat[:, :h])
    b = _encode_rows(mat[:, h:])
    return np.concatenate([a ^ b, b], axis=1)


_G = _encode_rows(np.eye(_N, dtype=np.int64))
_G_INFO = _G[_INFO_IDX, :].astype(np.float32)
_G_UHAT = _G[:, _INFO_IDX].astype(np.float32)


def _mod2(v):
    return v - 2.0 * jnp.floor(v * 0.5)


def _dot_t(a, b):
    return jax.lax.dot_general(a, b, (((0,), (0,)), ((), ())),
                               preferred_element_type=jnp.float32)


_SIGN = np.uint32(0x80000000)


def _u32(v):
    return jax.lax.bitcast_convert_type(v, jnp.uint32)


def _f32(v):
    return jax.lax.bitcast_convert_type(v, jnp.float32)


def _decode_x(llr, fr):
    if fr.all():
        return jnp.zeros(llr.shape, jnp.uint32)
    if not fr.any():
        return _u32(llr) & _SIGN
    n = fr.shape[0]
    k_info = n - int(fr.sum())
    if k_info == 1 and not fr[-1]:
        tot = jnp.sum(llr, axis=0, keepdims=True)
        return jnp.broadcast_to(_u32(tot) & _SIGN, llr.shape)
    if k_info == n - 1 and bool(fr[0]):
        hd = _u32(llr) & _SIGN
        par = jnp.sum((hd >> 31).astype(jnp.int32), axis=0, keepdims=True)
        par_mask = (((par & 1) << 31).astype(jnp.uint32))
        absl = jnp.abs(llr)
        mn = jnp.min(absl, axis=0, keepdims=True)
        flip = jnp.where(absl == mn,
                         jnp.broadcast_to(par_mask, llr.shape),
                         jnp.zeros(llr.shape, jnp.uint32))
        return hd ^ flip
    h = n // 2
    la, lb = llr[:h], llr[h:]
    lf, rf = fr[:h], fr[h:]
    if lf.all():
        x2 = _decode_x(la + lb, rf)
        return jnp.concatenate([x2, x2], axis=0)
    l1 = _f32((_u32(la) ^ _u32(lb)) & _SIGN
              | _u32(jnp.minimum(jnp.abs(la), jnp.abs(lb))))
    x1 = _decode_x(l1, lf)
    l2 = lb + _f32(_u32(la) ^ x1)
    x2 = _decode_x(l2, rf)
    return jnp.concatenate([x1 ^ x2, x2], axis=0)


def _fused_body(u_ref, a_ref, nr_ref, ni_ref, ebno_ref,
                w1_ref, b1_ref, w2x_ref, b2x_ref, w2p_ref, b2p_ref,
                gi_ref, gu_ref, id_ref,
                ct_ref, uh_ref, ch_ref, pa_ref, ah_ref, uo_ref, ao_ref):
    uo_ref[...] = u_ref[...]
    ao_ref[...] = a_ref[...]
    a_col = jnp.transpose(a_ref[...])

    ct = jax.lax.dot_general(u_ref[...], gi_ref[...], (((1,), (0,)), ((), ())),
                             preferred_element_type=jnp.float32)
    ct = _mod2(ct)
    ct_ref[...] = ct

    no = 1.0 / (jnp.exp((ebno_ref[0] / 10.0)
                        * np.float32(np.log(10.0))) * _RATE)
    s = jnp.sqrt(no / 2.0)
    y_r = (1.0 - 2.0 * ct) * a_col + nr_ref[...] * s
    y_i = ni_ref[...] * s
    y_ri = jnp.concatenate([y_r, y_i], axis=1)

    h = jnp.maximum(
        jax.lax.dot_general(y_ri, w1_ref[...], (((1,), (0,)), ((), ())),
                            preferred_element_type=jnp.float32)
        + b1_ref[...][None, :],
        0.0)
    logit_t = jax.lax.dot_general(jnp.transpose(w2p_ref[...]), h,
                                  (((0,), (1,)), ((), ())),
                                  preferred_element_type=jnp.float32)
    logit_t = logit_t + b2p_ref[0]
    p = jax.nn.sigmoid(logit_t)
    pa_ref[...] = p
    ah = (p > _THRESH).astype(jnp.float32)
    ah_ref[...] = ah

    yhr_t = jax.lax.dot_general(w2x_ref[...], h, (((0,), (1,)), ((), ())),
                                preferred_element_type=jnp.float32)
    yhr_t = yhr_t + jnp.transpose(b2x_ref[...][None, 0:_N])
    llr_t = 4.0 * yhr_t / no

    xs_root = _decode_x(llr_t, _FROZEN)
    x_root = (xs_root >> 31).astype(jnp.float32)

    xm = x_root * ah
    uh_ref[...] = _mod2(
        jax.lax.dot_general(xm, gu_ref[...], (((0,), (0,)), ((), ())),
                            preferred_element_type=jnp.float32))
    ch_ref[...] = jax.lax.dot_general(xm, id_ref[...], (((0,), (0,)), ((), ())),
                                      preferred_element_type=jnp.float32)


def kernel(u, a_true, noise_r, noise_i, ebno_db, W1, b1, W2x, b2x, W2p, b2p):
    out_shapes = [
        jax.ShapeDtypeStruct((_B, _N), jnp.float32),
        jax.ShapeDtypeStruct((_B, _K), jnp.float32),
        jax.ShapeDtypeStruct((_B, _N), jnp.float32),
        jax.ShapeDtypeStruct((1, _B), jnp.float32),
        jax.ShapeDtypeStruct((1, _B), jnp.float32),
        jax.ShapeDtypeStruct((_B, _K), jnp.float32),
        jax.ShapeDtypeStruct((1, _B), jnp.float32),
    ]
    ct, uh, ch, pa, ah, uo, ao = pl.pallas_call(
        _fused_body,
        grid=(1,),
        in_specs=[
            pl.BlockSpec((_B, _K), lambda i: (0, 0)),
            pl.BlockSpec((1, _B), lambda i: (0, 0)),
            pl.BlockSpec((_B, _N), lambda i: (0, 0)),
            pl.BlockSpec((_B, _N), lambda i: (0, 0)),
            pl.BlockSpec((1,), lambda i: (0,)),
            pl.BlockSpec((2 * _N, _HIDDEN), lambda i: (0, 0)),
            pl.BlockSpec((_HIDDEN,), lambda i: (0,)),
            pl.BlockSpec((_HIDDEN, _N), lambda i: (0, 0)),
            pl.BlockSpec((2 * _N,), lambda i: (0,)),
            pl.BlockSpec((1, _HIDDEN), lambda i: (0, 0)),
            pl.BlockSpec((1,), lambda i: (0,)),
            pl.BlockSpec((_K, _N), lambda i: (0, 0)),
            pl.BlockSpec((_N, _K), lambda i: (0, 0)),
            pl.BlockSpec((_N, _N), lambda i: (0, 0)),
        ],
        out_specs=[
            pl.BlockSpec((_B, _N), lambda i: (0, 0)),
            pl.BlockSpec((_B, _K), lambda i: (0, 0)),
            pl.BlockSpec((_B, _N), lambda i: (0, 0)),
            pl.BlockSpec((1, _B), lambda i: (0, 0)),
            pl.BlockSpec((1, _B), lambda i: (0, 0)),
            pl.BlockSpec((_B, _K), lambda i: (0, 0)),
            pl.BlockSpec((1, _B), lambda i: (0, 0)),
        ],
        out_shape=out_shapes,
    )(
        u,
        a_true.reshape(1, _B),
        noise_r,
        noise_i,
        ebno_db.reshape(1),
        W1,
        b1,
        W2x,
        b2x,
        W2p.reshape(1, _HIDDEN),
        b2p,
        jnp.asarray(_G_INFO),
        jnp.asarray(_G_UHAT),
        jnp.asarray(np.eye(_N, dtype=np.float32)),
    )
    return (uo, uh, ct, ch, ao.reshape(_B, 1), pa.reshape(_B, 1),
            ah.reshape(_B, 1))
